# Initial kernel scaffold; baseline (speedup 1.0000x reference)
import functools

import jax
import jax.numpy as jnp
from jax import lax
from jax.experimental import pallas as pl
from jax.experimental.pallas import tpu as pltpu
from jax.experimental.pallas import tpu_sc as plsc

N = 10000
NP = 10240
E = 320000
EP = 327680
D = 128
H = 256
G = 64

_MESH = plsc.VectorSubcoreMesh(core_axis_name="c", subcore_axis_name="s")

_EC = 2048
_EC2 = 128


def _deg_body(dst_hbm, ew_hbm, out_hbm, deg_v, idx_v, ewc_v, stage_sh):
    c = lax.axis_index("c")
    s = lax.axis_index("s")
    w = s * 2 + c

    def zero(i, _):
        deg_v[pl.ds(i * 16, 16)] = jnp.zeros((16,), jnp.float32)
        return 0

    lax.fori_loop(0, NP // 16, zero, 0)

    tile_base = pl.multiple_of(w * (EP // 32), 8)
    for k in range(EP // 32 // _EC):
        base = pl.multiple_of(tile_base + k * _EC, 8)
        pltpu.sync_copy(dst_hbm.at[pl.ds(base, _EC)], idx_v)
        pltpu.sync_copy(ew_hbm.at[pl.ds(base, _EC)], ewc_v)

        def scat(j, _):
            sl = pl.ds(j * 16, 16)
            plsc.addupdate_scatter(deg_v, [idx_v[sl]], ewc_v[sl])
            return 0

        lax.fori_loop(0, _EC // 16, scat, 0)

    pltpu.sync_copy(deg_v, stage_sh.at[s])
    plsc.subcore_barrier()

    col = pl.multiple_of(s * (NP // 16), 8)
    lax.fori_loop(0, NP // 16 // 16, zero, 0)
    for r in range(16):
        pltpu.sync_copy(stage_sh.at[r, pl.ds(col, NP // 16)],
                        ewc_v.at[pl.ds(0, NP // 16)])

        def acc(j, _):
            sl = pl.ds(j * 16, 16)
            deg_v[sl] = deg_v[sl] + ewc_v[sl]
            return 0

        lax.fori_loop(0, NP // 16 // 16, acc, 0)
    pltpu.sync_copy(deg_v.at[pl.ds(0, NP // 16)],
                    out_hbm.at[c, pl.ds(col, NP // 16)])


_deg_call = functools.partial(
    pl.kernel,
    out_type=jax.ShapeDtypeStruct((2, NP), jnp.float32),
    mesh=_MESH,
    scratch_types=[
        pltpu.VMEM((NP,), jnp.float32),
        pltpu.VMEM((_EC,), jnp.int32),
        pltpu.VMEM((_EC,), jnp.float32),
        pltpu.VMEM_SHARED((16, NP), jnp.float32),
    ],
)(_deg_body)


def _dinv_body(degp_hbm, dinv_hbm, a_v, b_v):
    c = lax.axis_index("c")
    s = lax.axis_index("s")
    w = s * 2 + c
    nt = NP // 32
    base = pl.multiple_of(w * nt, 8)
    pltpu.sync_copy(degp_hbm.at[0, pl.ds(base, nt)], a_v)
    pltpu.sync_copy(degp_hbm.at[1, pl.ds(base, nt)], b_v)

    def body(j, _):
        sl = pl.ds(j * 16, 16)
        d = a_v[sl] + b_v[sl] + 1.0
        i = plsc.bitcast(d, jnp.int32)
        i = jnp.int32(0x5F3759DF) - (i >> 1)
        y = plsc.bitcast(i, jnp.float32)
        for _unused in range(3):
            y = y * (1.5 - 0.5 * d * y * y)
        a_v[sl] = y
        return 0

    lax.fori_loop(0, nt // 16, body, 0)
    pltpu.sync_copy(a_v, dinv_hbm.at[pl.ds(base, nt)])


_dinv_call = functools.partial(
    pl.kernel,
    out_type=jax.ShapeDtypeStruct((NP,), jnp.float32),
    mesh=_MESH,
    scratch_types=[
        pltpu.VMEM((NP // 32,), jnp.float32),
        pltpu.VMEM((NP // 32,), jnp.float32),
    ],
)(_dinv_body)


def _make_agg(F2):
    RB = 160

    def body(flo_hbm, fhi_hbm, src_hbm, dst_hbm, ew_hbm, dinv_hbm,
             olo_hbm, ohi_hbm,
             dinv_v, src_v, dst_v, ewc_v, nrm_v, rows_v, acc_sh):
        c = lax.axis_index("c")
        s = lax.axis_index("s")
        pltpu.sync_copy(dinv_hbm, dinv_v)

        def run(f_hbm, o_hbm):
            for kb in range(640 // RB):
                rbase = pl.multiple_of(s * 640 + kb * RB, 8)
                pltpu.sync_copy(f_hbm.at[pl.ds(rbase, RB)],
                                rows_v.at[pl.ds(0, RB)])

                def init_scale(r, _):
                    dv = dinv_v[rbase + r]
                    s2 = dv * dv
                    for j in range(F2 // 16):
                        sl = pl.ds(j * 16, 16)
                        rows_v[r, sl] = rows_v[r, sl] * s2
                    return 0

                lax.fori_loop(0, RB, init_scale, 0)
                pltpu.sync_copy(rows_v.at[pl.ds(0, RB)],
                                acc_sh.at[pl.ds(rbase, RB)])
            plsc.subcore_barrier()

            ebase0 = s * (EP // 16)

            def chunk(k, _):
                base = pl.multiple_of(ebase0 + k * _EC2, 8)
                pltpu.sync_copy(src_hbm.at[pl.ds(base, _EC2)], src_v)
                pltpu.sync_copy(dst_hbm.at[pl.ds(base, _EC2)], dst_v)
                pltpu.sync_copy(ew_hbm.at[pl.ds(base, _EC2)], ewc_v)

                def nrm(j, _):
                    sl = pl.ds(j * 16, 16)
                    n1 = plsc.load_gather(dinv_v, [src_v[sl]])
                    n2 = plsc.load_gather(dinv_v, [dst_v[sl]])
                    nrm_v[sl] = n1 * ewc_v[sl] * n2
                    return 0

                lax.fori_loop(0, _EC2 // 16, nrm, 0)
                pltpu.sync_copy(f_hbm.at[src_v], rows_v)

                def scale(r, _):
                    sc = nrm_v[r]
                    for j in range(F2 // 16):
                        sl = pl.ds(j * 16, 16)
                        rows_v[r, sl] = rows_v[r, sl] * sc
                    return 0

                lax.fori_loop(0, _EC2, scale, 0)
                pltpu.sync_copy(rows_v, acc_sh.at[dst_v], add=True)
                return 0

            lax.fori_loop(0, EP // 16 // _EC2, chunk, 0)
            plsc.subcore_barrier()

            for kb in range(640 // RB):
                rbase = pl.multiple_of(s * 640 + kb * RB, 8)
                pltpu.sync_copy(acc_sh.at[pl.ds(rbase, RB)],
                                rows_v.at[pl.ds(0, RB)])
                pltpu.sync_copy(rows_v.at[pl.ds(0, RB)],
                                o_hbm.at[pl.ds(rbase, RB)])

        @pl.when(c == 0)
        def _c0():
            run(flo_hbm, olo_hbm)

        @pl.when(c == 1)
        def _c1():
            run(fhi_hbm, ohi_hbm)

    return functools.partial(
        pl.kernel,
        out_type=(jax.ShapeDtypeStruct((NP, F2), jnp.float32),
                  jax.ShapeDtypeStruct((NP, F2), jnp.float32)),
        mesh=_MESH,
        scratch_types=[
            pltpu.VMEM((NP,), jnp.float32),
            pltpu.VMEM((_EC2,), jnp.int32),
            pltpu.VMEM((_EC2,), jnp.int32),
            pltpu.VMEM((_EC2,), jnp.float32),
            pltpu.VMEM((_EC2,), jnp.float32),
            pltpu.VMEM((_EC2, F2), jnp.float32),
            pltpu.VMEM_SHARED((NP, F2), jnp.float32),
        ],
    )(body)


_agg64 = _make_agg(64)
_agg128 = _make_agg(128)


def _tc_layer1(agg_lo, agg_hi, w1a, w1b, b1r):
    blk = 256

    def body(alo, ahi, wa, wb, b_, olo, ohi):
        h = (jnp.dot(alo[...], wa[...], preferred_element_type=jnp.float32)
             + jnp.dot(ahi[...], wb[...], preferred_element_type=jnp.float32)
             + b_[...])
        h = jnp.maximum(h, 0.0)
        olo[...] = h[:, :H // 2]
        ohi[...] = h[:, H // 2:]

    return pl.pallas_call(
        body,
        grid=(NP // blk,),
        in_specs=[
            pl.BlockSpec((blk, D // 2), lambda i: (i, 0)),
            pl.BlockSpec((blk, D // 2), lambda i: (i, 0)),
            pl.BlockSpec((D // 2, H), lambda i: (0, 0)),
            pl.BlockSpec((D // 2, H), lambda i: (0, 0)),
            pl.BlockSpec((1, H), lambda i: (0, 0)),
        ],
        out_specs=[pl.BlockSpec((blk, H // 2), lambda i: (i, 0)),
                   pl.BlockSpec((blk, H // 2), lambda i: (i, 0))],
        out_shape=[jax.ShapeDtypeStruct((NP, H // 2), jnp.float32),
                   jax.ShapeDtypeStruct((NP, H // 2), jnp.float32)],
    )(agg_lo, agg_hi, w1a, w1b, b1r)


def _tc_head(agg_lo, agg_hi, w2a, w2b, b2r, batch2d, wf1, bf1r, wf2p, bf2r):
    blk = 256
    nb = NP // blk

    def body(alo, ahi, wa, wb, b_, bt, wf1_, bf1_, wf2_, bf2_,
             out_ref, sums, cnts):
        i = pl.program_id(0)

        @pl.when(i == 0)
        def _():
            sums[...] = jnp.zeros_like(sums)
            cnts[...] = jnp.zeros_like(cnts)

        h2 = (jnp.dot(alo[...], wa[...], preferred_element_type=jnp.float32)
              + jnp.dot(ahi[...], wb[...], preferred_element_type=jnp.float32)
              + b_[...])
        gids = lax.broadcasted_iota(jnp.int32, (G, blk), 0)
        oh = (gids == bt[...]).astype(jnp.float32)
        sums[...] += jnp.dot(oh, h2, preferred_element_type=jnp.float32)
        cnts[...] += jnp.broadcast_to(jnp.sum(oh, axis=1, keepdims=True),
                                      (G, 128))

        @pl.when(i == nb - 1)
        def _():
            cc = jnp.maximum(cnts[...][:, :1], 1.0)
            pooled = sums[...] / jnp.broadcast_to(cc, (G, H))
            z = jnp.maximum(
                jnp.dot(pooled, wf1_[...], preferred_element_type=jnp.float32)
                + bf1_[...], 0.0)
            out_ref[...] = (jnp.dot(z, wf2_[...],
                                    preferred_element_type=jnp.float32)
                            + bf2_[...])

    return pl.pallas_call(
        body,
        grid=(nb,),
        in_specs=[
            pl.BlockSpec((blk, H // 2), lambda i: (i, 0)),
            pl.BlockSpec((blk, H // 2), lambda i: (i, 0)),
            pl.BlockSpec((H // 2, H), lambda i: (0, 0)),
            pl.BlockSpec((H // 2, H), lambda i: (0, 0)),
            pl.BlockSpec((1, H), lambda i: (0, 0)),
            pl.BlockSpec((1, blk), lambda i: (i, 0)),
            pl.BlockSpec((H, 64), lambda i: (0, 0)),
            pl.BlockSpec((1, 64), lambda i: (0, 0)),
            pl.BlockSpec((64, 128), lambda i: (0, 0)),
            pl.BlockSpec((1, 128), lambda i: (0, 0)),
        ],
        out_specs=pl.BlockSpec((G, 128), lambda i: (0, 0)),
        out_shape=jax.ShapeDtypeStruct((G, 128), jnp.float32),
        scratch_shapes=[pltpu.VMEM((G, H), jnp.float32),
                        pltpu.VMEM((G, 128), jnp.float32)],
    )(agg_lo, agg_hi, w2a, w2b, b2r, batch2d, wf1, bf1r, wf2p, bf2r)


def kernel(x, edge_index, edge_weight, batch, W1, b1, W2, b2,
           Wf1, bf1, Wf2, bf2):
    src = edge_index[0]
    dst = edge_index[1]
    srcp = jnp.pad(src, (0, EP - E))
    dstp = jnp.pad(dst, (0, EP - E))
    ewp = jnp.pad(edge_weight, (0, EP - E))
    xp = jnp.pad(x, ((0, NP - N), (0, 0)))
    batch2d = jnp.pad(batch, (0, NP - N), constant_values=-1).reshape(
        NP // 256, 256)

    degp = _deg_call(dstp, ewp)
    dinv = _dinv_call(degp)

    agg1_lo, agg1_hi = _agg64(xp[:, :D // 2], xp[:, D // 2:],
                              srcp, dstp, ewp, dinv)
    h1_lo, h1_hi = _tc_layer1(agg1_lo, agg1_hi, W1[:D // 2], W1[D // 2:],
                              b1.reshape(1, H))
    agg2_lo, agg2_hi = _agg128(h1_lo, h1_hi, srcp, dstp, ewp, dinv)

    outp = _tc_head(agg2_lo, agg2_hi, W2[:H // 2], W2[H // 2:],
                    b2.reshape(1, H), batch2d, Wf1, bf1.reshape(1, 64),
                    jnp.pad(Wf2, ((0, 0), (0, 125))),
                    jnp.pad(bf2, (0, 125)).reshape(1, 128))
    return outp[:, :3]


# speedup vs baseline: 5.4139x; 5.4139x over previous
import functools

import jax
import jax.numpy as jnp
from jax import lax
from jax.experimental import pallas as pl
from jax.experimental.pallas import tpu as pltpu
from jax.experimental.pallas import tpu_sc as plsc

N = 10000
NP = 10240
E = 320000
EP = 327680
D = 128
H = 256
G = 64

_MESH = plsc.VectorSubcoreMesh(core_axis_name="c", subcore_axis_name="s")

_EC = 2048
_EC2 = 128


def _deg_body(dst_hbm, ew_hbm, out_hbm, deg_v, idx_v, ewc_v, stage_sh):
    c = lax.axis_index("c")
    s = lax.axis_index("s")
    w = s * 2 + c

    def zero(i, _):
        deg_v[pl.ds(i * 16, 16)] = jnp.zeros((16,), jnp.float32)
        return 0

    lax.fori_loop(0, NP // 16, zero, 0)

    tile_base = pl.multiple_of(w * (EP // 32), 8)
    for k in range(EP // 32 // _EC):
        base = pl.multiple_of(tile_base + k * _EC, 8)
        pltpu.sync_copy(dst_hbm.at[pl.ds(base, _EC)], idx_v)
        pltpu.sync_copy(ew_hbm.at[pl.ds(base, _EC)], ewc_v)

        def scat(j, _):
            sl = pl.ds(j * 16, 16)
            plsc.addupdate_scatter(deg_v, [idx_v[sl]], ewc_v[sl])
            return 0

        lax.fori_loop(0, _EC // 16, scat, 0)

    pltpu.sync_copy(deg_v, stage_sh.at[pl.ds(s * NP, NP)])
    plsc.subcore_barrier()

    col = pl.multiple_of(s * (NP // 16), 8)
    lax.fori_loop(0, NP // 16 // 16, zero, 0)
    for r in range(16):
        pltpu.sync_copy(stage_sh.at[pl.ds(r * NP + col, NP // 16)],
                        ewc_v.at[pl.ds(0, NP // 16)])

        def acc(j, _):
            sl = pl.ds(j * 16, 16)
            deg_v[sl] = deg_v[sl] + ewc_v[sl]
            return 0

        lax.fori_loop(0, NP // 16 // 16, acc, 0)
    pltpu.sync_copy(deg_v.at[pl.ds(0, NP // 16)],
                    out_hbm.at[pl.ds(c * NP + col, NP // 16)])


_deg_call = functools.partial(
    pl.kernel,
    out_type=jax.ShapeDtypeStruct((2 * NP,), jnp.float32),
    mesh=_MESH,
    compiler_params=pltpu.CompilerParams(needs_layout_passes=False),
    scratch_types=[
        pltpu.VMEM((NP,), jnp.float32),
        pltpu.VMEM((_EC,), jnp.int32),
        pltpu.VMEM((_EC,), jnp.float32),
        pltpu.VMEM_SHARED((16 * NP,), jnp.float32),
    ],
)(_deg_body)


def _dinv_body(degp_hbm, dinv_hbm, a_v, b_v):
    c = lax.axis_index("c")
    s = lax.axis_index("s")
    w = s * 2 + c
    nt = NP // 32
    base = pl.multiple_of(w * nt, 8)
    pltpu.sync_copy(degp_hbm.at[pl.ds(base, nt)], a_v)
    pltpu.sync_copy(degp_hbm.at[pl.ds(NP + base, nt)], b_v)

    def body(j, _):
        sl = pl.ds(j * 16, 16)
        d = a_v[sl] + b_v[sl] + 1.0
        i = plsc.bitcast(d, jnp.int32)
        i = jnp.int32(0x5F3759DF) - (i >> 1)
        y = plsc.bitcast(i, jnp.float32)
        for _unused in range(3):
            y = y * (1.5 - 0.5 * d * y * y)
        a_v[sl] = y
        return 0

    lax.fori_loop(0, nt // 16, body, 0)
    pltpu.sync_copy(a_v, dinv_hbm.at[pl.ds(base, nt)])


_dinv_call = functools.partial(
    pl.kernel,
    out_type=jax.ShapeDtypeStruct((NP,), jnp.float32),
    mesh=_MESH,
    compiler_params=pltpu.CompilerParams(needs_layout_passes=False),
    scratch_types=[
        pltpu.VMEM((NP // 32,), jnp.float32),
        pltpu.VMEM((NP // 32,), jnp.float32),
    ],
)(_dinv_body)


_RB = 128


def _zero_rows(rows_v, nrows, width):
    def z(r, _):
        for j in range(width // 16):
            rows_v[r, pl.ds(j * 16, 16)] = jnp.zeros((16,), jnp.float32)
        return 0

    lax.fori_loop(0, nrows, z, 0)


def _init_self_loop(f_hbm, dinv_v, rows_v, acc_sh, s, width):
    for kb in range(640 // _RB):
        rbase = pl.multiple_of(s * 640 + kb * _RB, 8)
        pltpu.sync_copy(f_hbm.at[pl.ds(rbase, _RB)], rows_v.at[pl.ds(0, _RB)])

        def init_scale(jj, _):
            dv = dinv_v[pl.ds(rbase + jj * 16, 16)]
            dv2 = dv * dv
            for r in range(16):
                s2 = dv2[r]
                row = jj * 16 + r
                for j in range(width // 16):
                    sl = pl.ds(j * 16, 16)
                    rows_v[row, sl] = rows_v[row, sl] * s2
            return 0

        lax.fori_loop(0, _RB // 16, init_scale, 0)
        pltpu.sync_copy(rows_v.at[pl.ds(0, _RB)], acc_sh.at[pl.ds(rbase, _RB)])


def _edge_loop(f_hbm, src_hbm, dst_hbm, ew_hbm, dinv_v,
               src_v, dst_v, ewc_v, nrm_v, rows_v, acc_sh,
               ebase0, nchunks, width):
    def chunk(k, _):
        base = pl.multiple_of(ebase0 + k * _EC2, 8)
        pltpu.sync_copy(src_hbm.at[pl.ds(base, _EC2)], src_v)
        pltpu.sync_copy(dst_hbm.at[pl.ds(base, _EC2)], dst_v)
        pltpu.sync_copy(ew_hbm.at[pl.ds(base, _EC2)], ewc_v)

        def nrm(j, _):
            sl = pl.ds(j * 16, 16)
            n1 = plsc.load_gather(dinv_v, [src_v[sl]])
            n2 = plsc.load_gather(dinv_v, [dst_v[sl]])
            nrm_v[sl] = n1 * ewc_v[sl] * n2
            return 0

        lax.fori_loop(0, _EC2 // 16, nrm, 0)
        pltpu.sync_copy(f_hbm.at[src_v], rows_v)

        def scale(jj, _):
            nv = nrm_v[pl.ds(jj * 16, 16)]
            for r in range(16):
                sc = nv[r]
                row = jj * 16 + r
                for j in range(width // 16):
                    sl = pl.ds(j * 16, 16)
                    rows_v[row, sl] = rows_v[row, sl] * sc
            return 0

        lax.fori_loop(0, _EC2 // 16, scale, 0)
        pltpu.sync_copy(rows_v, acc_sh.at[dst_v], add=True)
        return 0

    lax.fori_loop(0, nchunks, chunk, 0)


def _writeout(o_hbm, rows_v, acc_sh, s):
    for kb in range(640 // _RB):
        rbase = pl.multiple_of(s * 640 + kb * _RB, 8)
        pltpu.sync_copy(acc_sh.at[pl.ds(rbase, _RB)], rows_v.at[pl.ds(0, _RB)])
        pltpu.sync_copy(rows_v.at[pl.ds(0, _RB)], o_hbm.at[pl.ds(rbase, _RB)])


def _agg2_body(flo_hbm, fhi_hbm, src_hbm, dst_hbm, ew_hbm, dinv_hbm,
               olo_hbm, ohi_hbm,
               dinv_v, src_v, dst_v, ewc_v, nrm_v, rows_v, acc_sh):
    c = lax.axis_index("c")
    s = lax.axis_index("s")
    F2 = H // 2
    pltpu.sync_copy(dinv_hbm, dinv_v)

    def run(f_hbm, o_hbm):
        _init_self_loop(f_hbm, dinv_v, rows_v, acc_sh, s, F2)
        plsc.subcore_barrier()

        _edge_loop(f_hbm, src_hbm, dst_hbm, ew_hbm, dinv_v,
                   src_v, dst_v, ewc_v, nrm_v, rows_v, acc_sh,
                   s * (EP // 16), EP // 16 // _EC2, F2)
        plsc.subcore_barrier()
        _writeout(o_hbm, rows_v, acc_sh, s)

    @pl.when(c == 0)
    def _c0():
        run(flo_hbm, olo_hbm)

    @pl.when(c == 1)
    def _c1():
        run(fhi_hbm, ohi_hbm)


_agg2_call = functools.partial(
    pl.kernel,
    out_type=(jax.ShapeDtypeStruct((NP, H // 2), jnp.float32),
              jax.ShapeDtypeStruct((NP, H // 2), jnp.float32)),
    mesh=_MESH,
    compiler_params=pltpu.CompilerParams(needs_layout_passes=False),
    scratch_types=[
        pltpu.VMEM((NP,), jnp.float32),
        pltpu.VMEM((_EC2,), jnp.int32),
        pltpu.VMEM((_EC2,), jnp.int32),
        pltpu.VMEM((_EC2,), jnp.float32),
        pltpu.VMEM((_EC2,), jnp.float32),
        pltpu.VMEM((_EC2, H // 2), jnp.float32),
        pltpu.VMEM_SHARED((NP, H // 2), jnp.float32),
    ],
)(_agg2_body)


def _tc_mm1(xp, w1):
    blk = 256

    def body(x_, w_, olo, ohi):
        h = jnp.dot(x_[...], w_[...], preferred_element_type=jnp.float32,
                     precision=lax.Precision.HIGHEST)
        olo[...] = h[:, :H // 2]
        ohi[...] = h[:, H // 2:]

    return pl.pallas_call(
        body,
        grid=(NP // blk,),
        in_specs=[
            pl.BlockSpec((blk, D), lambda i: (i, 0)),
            pl.BlockSpec((D, H), lambda i: (0, 0)),
        ],
        out_specs=[pl.BlockSpec((blk, H // 2), lambda i: (i, 0)),
                   pl.BlockSpec((blk, H // 2), lambda i: (i, 0))],
        out_shape=[jax.ShapeDtypeStruct((NP, H // 2), jnp.float32),
                   jax.ShapeDtypeStruct((NP, H // 2), jnp.float32)],
    )(xp, w1)


def _tc_mm2(alo, ahi, b1r, w2):
    blk = 256

    def body(alo_, ahi_, b_, w_, olo, ohi):
        bv = b_[...]
        wv = w_[...]
        h1lo = jnp.maximum(alo_[...] + bv[:, :H // 2], 0.0)
        h1hi = jnp.maximum(ahi_[...] + bv[:, H // 2:], 0.0)
        h = (jnp.dot(h1lo, wv[:H // 2], preferred_element_type=jnp.float32,
                     precision=lax.Precision.HIGHEST)
             + jnp.dot(h1hi, wv[H // 2:], preferred_element_type=jnp.float32,
                     precision=lax.Precision.HIGHEST))
        olo[...] = h[:, :H // 2]
        ohi[...] = h[:, H // 2:]

    return pl.pallas_call(
        body,
        grid=(NP // blk,),
        in_specs=[
            pl.BlockSpec((blk, H // 2), lambda i: (i, 0)),
            pl.BlockSpec((blk, H // 2), lambda i: (i, 0)),
            pl.BlockSpec((1, H), lambda i: (0, 0)),
            pl.BlockSpec((H, H), lambda i: (0, 0)),
        ],
        out_specs=[pl.BlockSpec((blk, H // 2), lambda i: (i, 0)),
                   pl.BlockSpec((blk, H // 2), lambda i: (i, 0))],
        out_shape=[jax.ShapeDtypeStruct((NP, H // 2), jnp.float32),
                   jax.ShapeDtypeStruct((NP, H // 2), jnp.float32)],
    )(alo, ahi, b1r, w2)


def _tc_head(alo, ahi, b2r, batch2d, wf1, bf1r, wf2p, bf2r):
    blk = 256
    nb = NP // blk

    def body(alo_, ahi_, b_, bt, wf1_, bf1_, wf2_, bf2_,
             out_ref, sums_lo, sums_hi, cnts):
        i = pl.program_id(0)

        @pl.when(i == 0)
        def _():
            sums_lo[...] = jnp.zeros_like(sums_lo)
            sums_hi[...] = jnp.zeros_like(sums_hi)
            cnts[...] = jnp.zeros_like(cnts)

        bv = b_[...]
        h2lo = alo_[...] + bv[:, :H // 2]
        h2hi = ahi_[...] + bv[:, H // 2:]
        gids = lax.broadcasted_iota(jnp.int32, (G, blk), 0)
        oh = (gids == bt[...].reshape(1, blk)).astype(jnp.float32)
        hp = lax.Precision.HIGHEST
        sums_lo[...] += jnp.dot(oh, h2lo, preferred_element_type=jnp.float32,
                                precision=hp)
        sums_hi[...] += jnp.dot(oh, h2hi, preferred_element_type=jnp.float32,
                                precision=hp)
        cnts[...] += jnp.broadcast_to(jnp.sum(oh, axis=1, keepdims=True),
                                      (G, 128))

        @pl.when(i == nb - 1)
        def _():
            cc = jnp.broadcast_to(jnp.maximum(cnts[...][:, :1], 1.0),
                                  (G, H // 2))
            plo = sums_lo[...] / cc
            phi = sums_hi[...] / cc
            wf1v = wf1_[...]
            z = jnp.maximum(
                jnp.dot(plo, wf1v[:H // 2], preferred_element_type=jnp.float32,
                     precision=lax.Precision.HIGHEST)
                + jnp.dot(phi, wf1v[H // 2:],
                          preferred_element_type=jnp.float32,
                     precision=lax.Precision.HIGHEST)
                + bf1_[...], 0.0)
            out_ref[...] = (jnp.dot(z, wf2_[...],
                                    preferred_element_type=jnp.float32,
                     precision=lax.Precision.HIGHEST)
                            + bf2_[...])

    return pl.pallas_call(
        body,
        grid=(nb,),
        in_specs=[
            pl.BlockSpec((blk, H // 2), lambda i: (i, 0)),
            pl.BlockSpec((blk, H // 2), lambda i: (i, 0)),
            pl.BlockSpec((1, H), lambda i: (0, 0)),
            pl.BlockSpec((1, 1, blk), lambda i: (i, 0, 0)),
            pl.BlockSpec((H, 64), lambda i: (0, 0)),
            pl.BlockSpec((1, 64), lambda i: (0, 0)),
            pl.BlockSpec((64, 128), lambda i: (0, 0)),
            pl.BlockSpec((1, 128), lambda i: (0, 0)),
        ],
        out_specs=pl.BlockSpec((G, 128), lambda i: (0, 0)),
        out_shape=jax.ShapeDtypeStruct((G, 128), jnp.float32),
        scratch_shapes=[pltpu.VMEM((G, H // 2), jnp.float32),
                        pltpu.VMEM((G, H // 2), jnp.float32),
                        pltpu.VMEM((G, 128), jnp.float32)],
    )(alo, ahi, b2r, batch2d, wf1, bf1r, wf2p, bf2r)


def _dbg_agg_jnp(f_lo, f_hi, srcp, dstp, ewp, dinv):
    f = jnp.concatenate([f_lo, f_hi], axis=1)
    nrm = dinv[srcp] * ewp * dinv[dstp]
    msg = f[srcp] * nrm[:, None]
    out = jnp.zeros_like(f).at[dstp].add(msg)
    out = out + (dinv * dinv)[:, None] * f
    return out[:, :H // 2], out[:, H // 2:]


def kernel(x, edge_index, edge_weight, batch, W1, b1, W2, b2,
           Wf1, bf1, Wf2, bf2):
    src = edge_index[0]
    dst = edge_index[1]
    srcp = jnp.pad(src, (0, EP - E))
    dstp = jnp.pad(dst, (0, EP - E))
    ewp = jnp.pad(edge_weight, (0, EP - E))
    xp = jnp.pad(x, ((0, NP - N), (0, 0)))
    batch2d = jnp.pad(batch, (0, NP - N), constant_values=-1).reshape(
        NP // 256, 1, 256)

    degp = _deg_call(dstp, ewp)
    dinv = _dinv_call(degp)

    hx_lo, hx_hi = _tc_mm1(xp, W1)
    agg1_lo, agg1_hi = _agg2_call(hx_lo, hx_hi, srcp, dstp, ewp, dinv)
    hh_lo, hh_hi = _tc_mm2(agg1_lo, agg1_hi, b1.reshape(1, H), W2)
    agg2_lo, agg2_hi = _agg2_call(hh_lo, hh_hi, srcp, dstp, ewp, dinv)

    outp = _tc_head(agg2_lo, agg2_hi, b2.reshape(1, H), batch2d,
                    Wf1, bf1.reshape(1, 64),
                    jnp.pad(Wf2, ((0, 0), (0, 125))),
                    jnp.pad(bf2, (0, 125)).reshape(1, 128))
    return outp[:, :3]


# speedup vs baseline: 7.7371x; 1.4291x over previous
import functools

import jax
import jax.numpy as jnp
from jax import lax
from jax.experimental import pallas as pl
from jax.experimental.pallas import tpu as pltpu
from jax.experimental.pallas import tpu_sc as plsc

N = 10000
NP = 10240
E = 320000
EP = 327680
D = 128
H = 256
G = 64

_MESH = plsc.VectorSubcoreMesh(core_axis_name="c", subcore_axis_name="s")

_EC = 2048
_EC2 = 128


def _deg_body(dst_hbm, ew_hbm, out_hbm, deg_v, idx_v, ewc_v, stage_sh):
    c = lax.axis_index("c")
    s = lax.axis_index("s")
    w = s * 2 + c

    def zero(i, _):
        deg_v[pl.ds(i * 16, 16)] = jnp.zeros((16,), jnp.float32)
        return 0

    lax.fori_loop(0, NP // 16, zero, 0)

    tile_base = pl.multiple_of(w * (EP // 32), 8)
    for k in range(EP // 32 // _EC):
        base = pl.multiple_of(tile_base + k * _EC, 8)
        pltpu.sync_copy(dst_hbm.at[pl.ds(base, _EC)], idx_v)
        pltpu.sync_copy(ew_hbm.at[pl.ds(base, _EC)], ewc_v)

        def scat(j, _):
            sl = pl.ds(j * 16, 16)
            plsc.addupdate_scatter(deg_v, [idx_v[sl]], ewc_v[sl])
            return 0

        lax.fori_loop(0, _EC // 16, scat, 0)

    pltpu.sync_copy(deg_v, stage_sh.at[pl.ds(s * NP, NP)])
    plsc.subcore_barrier()

    col = pl.multiple_of(s * (NP // 16), 8)
    lax.fori_loop(0, NP // 16 // 16, zero, 0)
    for r in range(16):
        pltpu.sync_copy(stage_sh.at[pl.ds(r * NP + col, NP // 16)],
                        ewc_v.at[pl.ds(0, NP // 16)])

        def acc(j, _):
            sl = pl.ds(j * 16, 16)
            deg_v[sl] = deg_v[sl] + ewc_v[sl]
            return 0

        lax.fori_loop(0, NP // 16 // 16, acc, 0)
    pltpu.sync_copy(deg_v.at[pl.ds(0, NP // 16)],
                    out_hbm.at[pl.ds(c * NP + col, NP // 16)])


_deg_call = functools.partial(
    pl.kernel,
    out_type=jax.ShapeDtypeStruct((2 * NP,), jnp.float32),
    mesh=_MESH,
    compiler_params=pltpu.CompilerParams(needs_layout_passes=False),
    scratch_types=[
        pltpu.VMEM((NP,), jnp.float32),
        pltpu.VMEM((_EC,), jnp.int32),
        pltpu.VMEM((_EC,), jnp.float32),
        pltpu.VMEM_SHARED((16 * NP,), jnp.float32),
    ],
)(_deg_body)


def _dinv_body(degp_hbm, dinv_hbm, a_v, b_v):
    c = lax.axis_index("c")
    s = lax.axis_index("s")
    w = s * 2 + c
    nt = NP // 32
    base = pl.multiple_of(w * nt, 8)
    pltpu.sync_copy(degp_hbm.at[pl.ds(base, nt)], a_v)
    pltpu.sync_copy(degp_hbm.at[pl.ds(NP + base, nt)], b_v)

    def body(j, _):
        sl = pl.ds(j * 16, 16)
        d = a_v[sl] + b_v[sl] + 1.0
        i = plsc.bitcast(d, jnp.int32)
        i = jnp.int32(0x5F3759DF) - (i >> 1)
        y = plsc.bitcast(i, jnp.float32)
        for _unused in range(3):
            y = y * (1.5 - 0.5 * d * y * y)
        a_v[sl] = y
        return 0

    lax.fori_loop(0, nt // 16, body, 0)
    pltpu.sync_copy(a_v, dinv_hbm.at[pl.ds(base, nt)])


_dinv_call = functools.partial(
    pl.kernel,
    out_type=jax.ShapeDtypeStruct((NP,), jnp.float32),
    mesh=_MESH,
    compiler_params=pltpu.CompilerParams(needs_layout_passes=False),
    scratch_types=[
        pltpu.VMEM((NP // 32,), jnp.float32),
        pltpu.VMEM((NP // 32,), jnp.float32),
    ],
)(_dinv_body)


_RB = 128


def _zero_rows(rows_v, nrows, width):
    def z(r, _):
        for j in range(width // 16):
            rows_v[r, pl.ds(j * 16, 16)] = jnp.zeros((16,), jnp.float32)
        return 0

    lax.fori_loop(0, nrows, z, 0)


def _init_self_loop(f_hbm, dinv_v, rows_v, acc_sh, s, width):
    for kb in range(640 // _RB):
        rbase = pl.multiple_of(s * 640 + kb * _RB, 8)
        pltpu.sync_copy(f_hbm.at[pl.ds(rbase, _RB)], rows_v.at[pl.ds(0, _RB)])

        def init_scale(jj, _):
            dv = dinv_v[pl.ds(rbase + jj * 16, 16)]
            dv2 = dv * dv
            for r in range(16):
                s2 = dv2[r]
                row = jj * 16 + r
                for j in range(width // 16):
                    sl = pl.ds(j * 16, 16)
                    rows_v[row, sl] = rows_v[row, sl] * s2
            return 0

        lax.fori_loop(0, _RB // 16, init_scale, 0)
        pltpu.sync_copy(rows_v.at[pl.ds(0, _RB)], acc_sh.at[pl.ds(rbase, _RB)])


_SCN = 8


def _edge_loop(f_hbm, src2_hbm, dst2_hbm, ew_hbm, dinv_v,
               src2_v, dst2_v, nrm_v, rows_a, rows_b,
               sem_ga, sem_gb, sem_sa, sem_sb, acc_sh, s, width):
    nch = EP // 16 // _EC2
    nsc = nch // _SCN

    def scale(buf, kk):
        def scale16(jj, _):
            nv = nrm_v[pl.ds(kk * _EC2 + jj * 16, 16)]
            for r in range(16):
                sc = nv[r]
                row = jj * 16 + r
                for j in range(width // 16):
                    sl = pl.ds(j * 16, 16)
                    buf[row, sl] = buf[row, sl] * sc
            return 0

        lax.fori_loop(0, _EC2 // 16, scale16, 0)

    def gather(kk, buf, sem):
        pltpu.async_copy(f_hbm.at[src2_v.at[kk]], buf, sem)

    def scatter(kk, buf, sem):
        pltpu.async_copy(buf, acc_sh.at[dst2_v.at[kk]], sem, add=True)

    def wait_gather(buf, sem):
        pltpu.make_async_copy(f_hbm.at[src2_v.at[0]], buf, sem).wait()

    def wait_scatter(buf, sem):
        pltpu.make_async_copy(buf, acc_sh.at[dst2_v.at[0]], sem).wait()

    def superblock(u, _):
        crow = pl.multiple_of(s * nch + u * _SCN, 8)
        ebase = pl.multiple_of((s * nch + u * _SCN) * _EC2, 8)
        pltpu.sync_copy(src2_hbm.at[pl.ds(crow, _SCN)], src2_v)
        pltpu.sync_copy(dst2_hbm.at[pl.ds(crow, _SCN)], dst2_v)
        pltpu.sync_copy(ew_hbm.at[pl.ds(ebase, _SCN * _EC2)], nrm_v)

        def nrm_row(r, _):
            for j in range(_EC2 // 16):
                sl = pl.ds(j * 16, 16)
                fl = pl.ds(r * _EC2 + j * 16, 16)
                n1 = plsc.load_gather(dinv_v, [src2_v[r, sl]])
                n2 = plsc.load_gather(dinv_v, [dst2_v[r, sl]])
                nrm_v[fl] = n1 * nrm_v[fl] * n2
            return 0

        lax.fori_loop(0, _SCN, nrm_row, 0)

        gather(0, rows_a, sem_ga)

        def pipe(t, _):
            ka = 2 * t
            kb = 2 * t + 1

            @pl.when(t > 0)
            def _():
                wait_scatter(rows_b, sem_sb)

            gather(kb, rows_b, sem_gb)
            wait_gather(rows_a, sem_ga)
            scale(rows_a, ka)
            scatter(ka, rows_a, sem_sa)

            @pl.when(t < _SCN // 2 - 1)
            def _():
                wait_scatter(rows_a, sem_sa)
                gather(ka + 2, rows_a, sem_ga)

            wait_gather(rows_b, sem_gb)
            scale(rows_b, kb)
            scatter(kb, rows_b, sem_sb)
            return 0

        lax.fori_loop(0, _SCN // 2, pipe, 0)
        wait_scatter(rows_a, sem_sa)
        wait_scatter(rows_b, sem_sb)
        return 0

    lax.fori_loop(0, nsc, superblock, 0)


def _writeout(o_hbm, rows_v, acc_sh, s):
    for kb in range(640 // _RB):
        rbase = pl.multiple_of(s * 640 + kb * _RB, 8)
        pltpu.sync_copy(acc_sh.at[pl.ds(rbase, _RB)], rows_v.at[pl.ds(0, _RB)])
        pltpu.sync_copy(rows_v.at[pl.ds(0, _RB)], o_hbm.at[pl.ds(rbase, _RB)])


def _agg2_body(flo_hbm, fhi_hbm, src2_hbm, dst2_hbm, ew_hbm, dinv_hbm,
               olo_hbm, ohi_hbm,
               dinv_v, src2_v, dst2_v, nrm_v, rows_a, rows_b,
               sem_ga, sem_gb, sem_sa, sem_sb, acc_sh):
    c = lax.axis_index("c")
    s = lax.axis_index("s")
    F2 = H // 2
    pltpu.sync_copy(dinv_hbm, dinv_v)

    def run(f_hbm, o_hbm):
        _init_self_loop(f_hbm, dinv_v, rows_a, acc_sh, s, F2)
        plsc.subcore_barrier()
        _edge_loop(f_hbm, src2_hbm, dst2_hbm, ew_hbm, dinv_v,
                   src2_v, dst2_v, nrm_v, rows_a, rows_b,
                   sem_ga, sem_gb, sem_sa, sem_sb, acc_sh, s, F2)
        plsc.subcore_barrier()
        _writeout(o_hbm, rows_a, acc_sh, s)

    @pl.when(c == 0)
    def _c0():
        run(flo_hbm, olo_hbm)

    @pl.when(c == 1)
    def _c1():
        run(fhi_hbm, ohi_hbm)


_agg2_call = functools.partial(
    pl.kernel,
    out_type=(jax.ShapeDtypeStruct((NP, H // 2), jnp.float32),
              jax.ShapeDtypeStruct((NP, H // 2), jnp.float32)),
    mesh=_MESH,
    compiler_params=pltpu.CompilerParams(needs_layout_passes=False),
    scratch_types=[
        pltpu.VMEM((NP,), jnp.float32),
        pltpu.VMEM((_SCN, _EC2), jnp.int32),
        pltpu.VMEM((_SCN, _EC2), jnp.int32),
        pltpu.VMEM((_SCN * _EC2,), jnp.float32),
        pltpu.VMEM((_EC2, H // 2), jnp.float32),
        pltpu.VMEM((_EC2, H // 2), jnp.float32),
        pltpu.SemaphoreType.DMA,
        pltpu.SemaphoreType.DMA,
        pltpu.SemaphoreType.DMA,
        pltpu.SemaphoreType.DMA,
        pltpu.VMEM_SHARED((NP, H // 2), jnp.float32),
    ],
)(_agg2_body)


def _tc_mm1(xp, w1):
    blk = 256

    def body(x_, w_, olo, ohi):
        h = jnp.dot(x_[...], w_[...], preferred_element_type=jnp.float32,
                     precision=lax.Precision.HIGHEST)
        olo[...] = h[:, :H // 2]
        ohi[...] = h[:, H // 2:]

    return pl.pallas_call(
        body,
        grid=(NP // blk,),
        in_specs=[
            pl.BlockSpec((blk, D), lambda i: (i, 0)),
            pl.BlockSpec((D, H), lambda i: (0, 0)),
        ],
        out_specs=[pl.BlockSpec((blk, H // 2), lambda i: (i, 0)),
                   pl.BlockSpec((blk, H // 2), lambda i: (i, 0))],
        out_shape=[jax.ShapeDtypeStruct((NP, H // 2), jnp.float32),
                   jax.ShapeDtypeStruct((NP, H // 2), jnp.float32)],
    )(xp, w1)


def _tc_mm2(alo, ahi, b1r, w2):
    blk = 256

    def body(alo_, ahi_, b_, w_, olo, ohi):
        bv = b_[...]
        wv = w_[...]
        h1lo = jnp.maximum(alo_[...] + bv[:, :H // 2], 0.0)
        h1hi = jnp.maximum(ahi_[...] + bv[:, H // 2:], 0.0)
        h = (jnp.dot(h1lo, wv[:H // 2], preferred_element_type=jnp.float32,
                     precision=lax.Precision.HIGHEST)
             + jnp.dot(h1hi, wv[H // 2:], preferred_element_type=jnp.float32,
                     precision=lax.Precision.HIGHEST))
        olo[...] = h[:, :H // 2]
        ohi[...] = h[:, H // 2:]

    return pl.pallas_call(
        body,
        grid=(NP // blk,),
        in_specs=[
            pl.BlockSpec((blk, H // 2), lambda i: (i, 0)),
            pl.BlockSpec((blk, H // 2), lambda i: (i, 0)),
            pl.BlockSpec((1, H), lambda i: (0, 0)),
            pl.BlockSpec((H, H), lambda i: (0, 0)),
        ],
        out_specs=[pl.BlockSpec((blk, H // 2), lambda i: (i, 0)),
                   pl.BlockSpec((blk, H // 2), lambda i: (i, 0))],
        out_shape=[jax.ShapeDtypeStruct((NP, H // 2), jnp.float32),
                   jax.ShapeDtypeStruct((NP, H // 2), jnp.float32)],
    )(alo, ahi, b1r, w2)


def _tc_head(alo, ahi, b2r, batch2d, wf1, bf1r, wf2p, bf2r):
    blk = 256
    nb = NP // blk

    def body(alo_, ahi_, b_, bt, wf1_, bf1_, wf2_, bf2_,
             out_ref, sums_lo, sums_hi, cnts):
        i = pl.program_id(0)

        @pl.when(i == 0)
        def _():
            sums_lo[...] = jnp.zeros_like(sums_lo)
            sums_hi[...] = jnp.zeros_like(sums_hi)
            cnts[...] = jnp.zeros_like(cnts)

        bv = b_[...]
        h2lo = alo_[...] + bv[:, :H // 2]
        h2hi = ahi_[...] + bv[:, H // 2:]
        gids = lax.broadcasted_iota(jnp.int32, (G, blk), 0)
        oh = (gids == bt[...].reshape(1, blk)).astype(jnp.float32)
        hp = lax.Precision.HIGHEST
        sums_lo[...] += jnp.dot(oh, h2lo, preferred_element_type=jnp.float32,
                                precision=hp)
        sums_hi[...] += jnp.dot(oh, h2hi, preferred_element_type=jnp.float32,
                                precision=hp)
        cnts[...] += jnp.broadcast_to(jnp.sum(oh, axis=1, keepdims=True),
                                      (G, 128))

        @pl.when(i == nb - 1)
        def _():
            cc = jnp.broadcast_to(jnp.maximum(cnts[...][:, :1], 1.0),
                                  (G, H // 2))
            plo = sums_lo[...] / cc
            phi = sums_hi[...] / cc
            wf1v = wf1_[...]
            z = jnp.maximum(
                jnp.dot(plo, wf1v[:H // 2], preferred_element_type=jnp.float32,
                     precision=lax.Precision.HIGHEST)
                + jnp.dot(phi, wf1v[H // 2:],
                          preferred_element_type=jnp.float32,
                     precision=lax.Precision.HIGHEST)
                + bf1_[...], 0.0)
            out_ref[...] = (jnp.dot(z, wf2_[...],
                                    preferred_element_type=jnp.float32,
                     precision=lax.Precision.HIGHEST)
                            + bf2_[...])

    return pl.pallas_call(
        body,
        grid=(nb,),
        in_specs=[
            pl.BlockSpec((blk, H // 2), lambda i: (i, 0)),
            pl.BlockSpec((blk, H // 2), lambda i: (i, 0)),
            pl.BlockSpec((1, H), lambda i: (0, 0)),
            pl.BlockSpec((1, 1, blk), lambda i: (i, 0, 0)),
            pl.BlockSpec((H, 64), lambda i: (0, 0)),
            pl.BlockSpec((1, 64), lambda i: (0, 0)),
            pl.BlockSpec((64, 128), lambda i: (0, 0)),
            pl.BlockSpec((1, 128), lambda i: (0, 0)),
        ],
        out_specs=pl.BlockSpec((G, 128), lambda i: (0, 0)),
        out_shape=jax.ShapeDtypeStruct((G, 128), jnp.float32),
        scratch_shapes=[pltpu.VMEM((G, H // 2), jnp.float32),
                        pltpu.VMEM((G, H // 2), jnp.float32),
                        pltpu.VMEM((G, 128), jnp.float32)],
    )(alo, ahi, b2r, batch2d, wf1, bf1r, wf2p, bf2r)


def _dbg_agg_jnp(f_lo, f_hi, srcp, dstp, ewp, dinv):
    f = jnp.concatenate([f_lo, f_hi], axis=1)
    nrm = dinv[srcp] * ewp * dinv[dstp]
    msg = f[srcp] * nrm[:, None]
    out = jnp.zeros_like(f).at[dstp].add(msg)
    out = out + (dinv * dinv)[:, None] * f
    return out[:, :H // 2], out[:, H // 2:]


def kernel(x, edge_index, edge_weight, batch, W1, b1, W2, b2,
           Wf1, bf1, Wf2, bf2):
    src = edge_index[0]
    dst = edge_index[1]
    srcp = jnp.pad(src, (0, EP - E))
    dstp = jnp.pad(dst, (0, EP - E))
    src2 = srcp.reshape(EP // _EC2, _EC2)
    dst2 = dstp.reshape(EP // _EC2, _EC2)
    ewp = jnp.pad(edge_weight, (0, EP - E))
    xp = jnp.pad(x, ((0, NP - N), (0, 0)))
    batch2d = jnp.pad(batch, (0, NP - N), constant_values=-1).reshape(
        NP // 256, 1, 256)

    degp = _deg_call(dstp, ewp)
    dinv = _dinv_call(degp)

    hx_lo, hx_hi = _tc_mm1(xp, W1)
    agg1_lo, agg1_hi = _agg2_call(hx_lo, hx_hi, src2, dst2, ewp, dinv)
    hh_lo, hh_hi = _tc_mm2(agg1_lo, agg1_hi, b1.reshape(1, H), W2)
    agg2_lo, agg2_hi = _agg2_call(hh_lo, hh_hi, src2, dst2, ewp, dinv)

    outp = _tc_head(agg2_lo, agg2_hi, b2.reshape(1, H), batch2d,
                    Wf1, bf1.reshape(1, 64),
                    jnp.pad(Wf2, ((0, 0), (0, 125))),
                    jnp.pad(bf2, (0, 125)).reshape(1, 128))
    return outp[:, :3]


# speedup vs baseline: 8.6626x; 1.1196x over previous
import functools

import jax
import jax.numpy as jnp
from jax import lax
from jax.experimental import pallas as pl
from jax.experimental.pallas import tpu as pltpu
from jax.experimental.pallas import tpu_sc as plsc

N = 10000
NP = 10240
E = 320000
EP = 327680
D = 128
H = 256
G = 64

_MESH = plsc.VectorSubcoreMesh(core_axis_name="c", subcore_axis_name="s")

_EC = 2048
_EC2 = 128


def _deg_body(dst_hbm, ew_hbm, out_hbm, deg_v, idx_v, ewc_v, stage_sh):
    c = lax.axis_index("c")
    s = lax.axis_index("s")
    w = s * 2 + c

    def zero(i, _):
        deg_v[pl.ds(i * 16, 16)] = jnp.zeros((16,), jnp.float32)
        return 0

    lax.fori_loop(0, NP // 16, zero, 0)

    tile_base = pl.multiple_of(w * (EP // 32), 8)
    for k in range(EP // 32 // _EC):
        base = pl.multiple_of(tile_base + k * _EC, 8)
        pltpu.sync_copy(dst_hbm.at[pl.ds(base, _EC)], idx_v)
        pltpu.sync_copy(ew_hbm.at[pl.ds(base, _EC)], ewc_v)

        def scat(j, _):
            sl = pl.ds(j * 16, 16)
            plsc.addupdate_scatter(deg_v, [idx_v[sl]], ewc_v[sl])
            return 0

        lax.fori_loop(0, _EC // 16, scat, 0)

    pltpu.sync_copy(deg_v, stage_sh.at[pl.ds(s * NP, NP)])
    plsc.subcore_barrier()

    col = pl.multiple_of(s * (NP // 16), 8)
    lax.fori_loop(0, NP // 16 // 16, zero, 0)
    for r in range(16):
        pltpu.sync_copy(stage_sh.at[pl.ds(r * NP + col, NP // 16)],
                        ewc_v.at[pl.ds(0, NP // 16)])

        def acc(j, _):
            sl = pl.ds(j * 16, 16)
            deg_v[sl] = deg_v[sl] + ewc_v[sl]
            return 0

        lax.fori_loop(0, NP // 16 // 16, acc, 0)
    pltpu.sync_copy(deg_v.at[pl.ds(0, NP // 16)],
                    out_hbm.at[pl.ds(c * NP + col, NP // 16)])


_deg_call = functools.partial(
    pl.kernel,
    out_type=jax.ShapeDtypeStruct((2 * NP,), jnp.float32),
    mesh=_MESH,
    compiler_params=pltpu.CompilerParams(needs_layout_passes=False),
    scratch_types=[
        pltpu.VMEM((NP,), jnp.float32),
        pltpu.VMEM((_EC,), jnp.int32),
        pltpu.VMEM((_EC,), jnp.float32),
        pltpu.VMEM_SHARED((16 * NP,), jnp.float32),
    ],
)(_deg_body)


def _dinv_body(degp_hbm, dinv_hbm, a_v, b_v):
    c = lax.axis_index("c")
    s = lax.axis_index("s")
    w = s * 2 + c
    nt = NP // 32
    base = pl.multiple_of(w * nt, 8)
    pltpu.sync_copy(degp_hbm.at[pl.ds(base, nt)], a_v)
    pltpu.sync_copy(degp_hbm.at[pl.ds(NP + base, nt)], b_v)

    def body(j, _):
        sl = pl.ds(j * 16, 16)
        d = a_v[sl] + b_v[sl] + 1.0
        i = plsc.bitcast(d, jnp.int32)
        i = jnp.int32(0x5F3759DF) - (i >> 1)
        y = plsc.bitcast(i, jnp.float32)
        for _unused in range(3):
            y = y * (1.5 - 0.5 * d * y * y)
        a_v[sl] = y
        return 0

    lax.fori_loop(0, nt // 16, body, 0)
    pltpu.sync_copy(a_v, dinv_hbm.at[pl.ds(base, nt)])


_dinv_call = functools.partial(
    pl.kernel,
    out_type=jax.ShapeDtypeStruct((NP,), jnp.float32),
    mesh=_MESH,
    compiler_params=pltpu.CompilerParams(needs_layout_passes=False),
    scratch_types=[
        pltpu.VMEM((NP // 32,), jnp.float32),
        pltpu.VMEM((NP // 32,), jnp.float32),
    ],
)(_dinv_body)


_RB = 128


def _zero_rows(rows_v, nrows, width):
    def z(r, _):
        for j in range(width // 16):
            rows_v[r, pl.ds(j * 16, 16)] = jnp.zeros((16,), jnp.float32)
        return 0

    lax.fori_loop(0, nrows, z, 0)


def _init_self_loop(f_hbm, dinv_v, rows_v, acc_sh, s, width):
    for kb in range(640 // _RB):
        rbase = pl.multiple_of(s * 640 + kb * _RB, 8)
        pltpu.sync_copy(f_hbm.at[pl.ds(rbase, _RB)], rows_v.at[pl.ds(0, _RB)])

        def init_scale(jj, _):
            dv = dinv_v[pl.ds(rbase + jj * 16, 16)]
            dv2 = dv * dv
            for r in range(16):
                s2 = dv2[r]
                row = jj * 16 + r
                for j in range(width // 16):
                    sl = pl.ds(j * 16, 16)
                    rows_v[row, sl] = rows_v[row, sl] * s2
            return 0

        lax.fori_loop(0, _RB // 16, init_scale, 0)
        pltpu.sync_copy(rows_v.at[pl.ds(0, _RB)], acc_sh.at[pl.ds(rbase, _RB)])


_SCN = 8


def _edge_loop(f_hbm, src2_hbm, dst2_hbm, ew_hbm, dinv_v,
               src2_v, dst2_v, nrm_v, rows_a, rows_b,
               sem_ga, sem_gb, sem_sa, sem_sb, acc_sh, crow0, nch, width):
    nsc = nch // _SCN

    def scale(buf, kk):
        def scale16(jj, _):
            nv = nrm_v[pl.ds(kk * _EC2 + jj * 16, 16)]
            for r in range(16):
                sc = nv[r]
                row = jj * 16 + r
                for j in range(width // 16):
                    sl = pl.ds(j * 16, 16)
                    buf[row, sl] = buf[row, sl] * sc
            return 0

        lax.fori_loop(0, _EC2 // 16, scale16, 0)

    def gather(kk, buf, sem):
        pltpu.async_copy(f_hbm.at[src2_v.at[kk]], buf, sem)

    def scatter(kk, buf, sem):
        pltpu.async_copy(buf, acc_sh.at[dst2_v.at[kk]], sem, add=True)

    def wait_gather(buf, sem):
        pltpu.make_async_copy(f_hbm.at[src2_v.at[0]], buf, sem).wait()

    def wait_scatter(buf, sem):
        pltpu.make_async_copy(buf, acc_sh.at[dst2_v.at[0]], sem).wait()

    def superblock(u, _):
        crow = pl.multiple_of(crow0 + u * _SCN, 8)
        ebase = pl.multiple_of((crow0 + u * _SCN) * _EC2, 8)
        pltpu.sync_copy(src2_hbm.at[pl.ds(crow, _SCN)], src2_v)
        pltpu.sync_copy(dst2_hbm.at[pl.ds(crow, _SCN)], dst2_v)
        pltpu.sync_copy(ew_hbm.at[pl.ds(ebase, _SCN * _EC2)], nrm_v)

        def nrm_row(r, _):
            for j in range(_EC2 // 16):
                sl = pl.ds(j * 16, 16)
                fl = pl.ds(r * _EC2 + j * 16, 16)
                n1 = plsc.load_gather(dinv_v, [src2_v[r, sl]])
                n2 = plsc.load_gather(dinv_v, [dst2_v[r, sl]])
                nrm_v[fl] = n1 * nrm_v[fl] * n2
            return 0

        lax.fori_loop(0, _SCN, nrm_row, 0)

        gather(0, rows_a, sem_ga)

        def pipe(t, _):
            ka = 2 * t
            kb = 2 * t + 1

            @pl.when(t > 0)
            def _():
                wait_scatter(rows_b, sem_sb)

            gather(kb, rows_b, sem_gb)
            wait_gather(rows_a, sem_ga)
            scale(rows_a, ka)
            scatter(ka, rows_a, sem_sa)

            @pl.when(t < _SCN // 2 - 1)
            def _():
                wait_scatter(rows_a, sem_sa)
                gather(ka + 2, rows_a, sem_ga)

            wait_gather(rows_b, sem_gb)
            scale(rows_b, kb)
            scatter(kb, rows_b, sem_sb)
            return 0

        lax.fori_loop(0, _SCN // 2, pipe, 0)
        wait_scatter(rows_a, sem_sa)
        wait_scatter(rows_b, sem_sb)
        return 0

    lax.fori_loop(0, nsc, superblock, 0)


def _writeout(o_hbm, rows_v, acc_sh, s):
    for kb in range(640 // _RB):
        rbase = pl.multiple_of(s * 640 + kb * _RB, 8)
        pltpu.sync_copy(acc_sh.at[pl.ds(rbase, _RB)], rows_v.at[pl.ds(0, _RB)])
        pltpu.sync_copy(rows_v.at[pl.ds(0, _RB)], o_hbm.at[pl.ds(rbase, _RB)])


def _agg1_body(f_hbm, src2_hbm, dst2_hbm, ew_hbm, dinv_hbm,
               o0_hbm, o1_hbm,
               dinv_v, src2_v, dst2_v, nrm_v, rows_a, rows_b,
               sem_ga, sem_gb, sem_sa, sem_sb, acc_sh):
    c = lax.axis_index("c")
    s = lax.axis_index("s")
    pltpu.sync_copy(dinv_hbm, dinv_v)

    @pl.when(c == 0)
    def _():
        _init_self_loop(f_hbm, dinv_v, rows_a, acc_sh, s, D)

    @pl.when(c == 1)
    def _():
        _zero_rows(rows_a, _RB, D)
        for kb in range(640 // _RB):
            rbase = pl.multiple_of(s * 640 + kb * _RB, 8)
            pltpu.sync_copy(rows_a.at[pl.ds(0, _RB)],
                            acc_sh.at[pl.ds(rbase, _RB)])

    plsc.subcore_barrier()
    nch1 = EP // 32 // _EC2
    _edge_loop(f_hbm, src2_hbm, dst2_hbm, ew_hbm, dinv_v,
               src2_v, dst2_v, nrm_v, rows_a, rows_b,
               sem_ga, sem_gb, sem_sa, sem_sb, acc_sh,
               (c * 16 + s) * nch1, nch1, D)
    plsc.subcore_barrier()

    @pl.when(c == 0)
    def _():
        _writeout(o0_hbm, rows_a, acc_sh, s)

    @pl.when(c == 1)
    def _():
        _writeout(o1_hbm, rows_a, acc_sh, s)


_agg1_call = functools.partial(
    pl.kernel,
    out_type=(jax.ShapeDtypeStruct((NP, D), jnp.float32),
              jax.ShapeDtypeStruct((NP, D), jnp.float32)),
    mesh=_MESH,
    compiler_params=pltpu.CompilerParams(needs_layout_passes=False),
    scratch_types=[
        pltpu.VMEM((NP,), jnp.float32),
        pltpu.VMEM((_SCN, _EC2), jnp.int32),
        pltpu.VMEM((_SCN, _EC2), jnp.int32),
        pltpu.VMEM((_SCN * _EC2,), jnp.float32),
        pltpu.VMEM((_EC2, D), jnp.float32),
        pltpu.VMEM((_EC2, D), jnp.float32),
        pltpu.SemaphoreType.DMA,
        pltpu.SemaphoreType.DMA,
        pltpu.SemaphoreType.DMA,
        pltpu.SemaphoreType.DMA,
        pltpu.VMEM_SHARED((NP, D), jnp.float32),
    ],
)(_agg1_body)


def _agg2_body(flo_hbm, fhi_hbm, src2_hbm, dst2_hbm, ew_hbm, dinv_hbm,
               olo_hbm, ohi_hbm,
               dinv_v, src2_v, dst2_v, nrm_v, rows_a, rows_b,
               sem_ga, sem_gb, sem_sa, sem_sb, acc_sh):
    c = lax.axis_index("c")
    s = lax.axis_index("s")
    F2 = H // 2
    pltpu.sync_copy(dinv_hbm, dinv_v)

    def run(f_hbm, o_hbm):
        _init_self_loop(f_hbm, dinv_v, rows_a, acc_sh, s, F2)
        plsc.subcore_barrier()
        _edge_loop(f_hbm, src2_hbm, dst2_hbm, ew_hbm, dinv_v,
                   src2_v, dst2_v, nrm_v, rows_a, rows_b,
                   sem_ga, sem_gb, sem_sa, sem_sb, acc_sh,
                   s * (EP // 16 // _EC2), EP // 16 // _EC2, F2)
        plsc.subcore_barrier()
        _writeout(o_hbm, rows_a, acc_sh, s)

    @pl.when(c == 0)
    def _c0():
        run(flo_hbm, olo_hbm)

    @pl.when(c == 1)
    def _c1():
        run(fhi_hbm, ohi_hbm)


_agg2_call = functools.partial(
    pl.kernel,
    out_type=(jax.ShapeDtypeStruct((NP, H // 2), jnp.float32),
              jax.ShapeDtypeStruct((NP, H // 2), jnp.float32)),
    mesh=_MESH,
    compiler_params=pltpu.CompilerParams(needs_layout_passes=False),
    scratch_types=[
        pltpu.VMEM((NP,), jnp.float32),
        pltpu.VMEM((_SCN, _EC2), jnp.int32),
        pltpu.VMEM((_SCN, _EC2), jnp.int32),
        pltpu.VMEM((_SCN * _EC2,), jnp.float32),
        pltpu.VMEM((_EC2, H // 2), jnp.float32),
        pltpu.VMEM((_EC2, H // 2), jnp.float32),
        pltpu.SemaphoreType.DMA,
        pltpu.SemaphoreType.DMA,
        pltpu.SemaphoreType.DMA,
        pltpu.SemaphoreType.DMA,
        pltpu.VMEM_SHARED((NP, H // 2), jnp.float32),
    ],
)(_agg2_body)


def _tc_mm1(a0, a1, w1):
    blk = 256

    def body(a0_, a1_, w_, olo, ohi):
        h = jnp.dot(a0_[...] + a1_[...], w_[...],
                    preferred_element_type=jnp.float32,
                    precision=lax.Precision.HIGHEST)
        olo[...] = h[:, :H // 2]
        ohi[...] = h[:, H // 2:]

    return pl.pallas_call(
        body,
        grid=(NP // blk,),
        in_specs=[
            pl.BlockSpec((blk, D), lambda i: (i, 0)),
            pl.BlockSpec((blk, D), lambda i: (i, 0)),
            pl.BlockSpec((D, H), lambda i: (0, 0)),
        ],
        out_specs=[pl.BlockSpec((blk, H // 2), lambda i: (i, 0)),
                   pl.BlockSpec((blk, H // 2), lambda i: (i, 0))],
        out_shape=[jax.ShapeDtypeStruct((NP, H // 2), jnp.float32),
                   jax.ShapeDtypeStruct((NP, H // 2), jnp.float32)],
    )(a0, a1, w1)


def _tc_mm2(alo, ahi, b1r, w2):
    blk = 256

    def body(alo_, ahi_, b_, w_, olo, ohi):
        bv = b_[...]
        wv = w_[...]
        h1lo = jnp.maximum(alo_[...] + bv[:, :H // 2], 0.0)
        h1hi = jnp.maximum(ahi_[...] + bv[:, H // 2:], 0.0)
        h = (jnp.dot(h1lo, wv[:H // 2], preferred_element_type=jnp.float32,
                     precision=lax.Precision.HIGHEST)
             + jnp.dot(h1hi, wv[H // 2:], preferred_element_type=jnp.float32,
                     precision=lax.Precision.HIGHEST))
        olo[...] = h[:, :H // 2]
        ohi[...] = h[:, H // 2:]

    return pl.pallas_call(
        body,
        grid=(NP // blk,),
        in_specs=[
            pl.BlockSpec((blk, H // 2), lambda i: (i, 0)),
            pl.BlockSpec((blk, H // 2), lambda i: (i, 0)),
            pl.BlockSpec((1, H), lambda i: (0, 0)),
            pl.BlockSpec((H, H), lambda i: (0, 0)),
        ],
        out_specs=[pl.BlockSpec((blk, H // 2), lambda i: (i, 0)),
                   pl.BlockSpec((blk, H // 2), lambda i: (i, 0))],
        out_shape=[jax.ShapeDtypeStruct((NP, H // 2), jnp.float32),
                   jax.ShapeDtypeStruct((NP, H // 2), jnp.float32)],
    )(alo, ahi, b1r, w2)


def _tc_head(alo, ahi, b2r, batch2d, wf1, bf1r, wf2p, bf2r):
    blk = 256
    nb = NP // blk

    def body(alo_, ahi_, b_, bt, wf1_, bf1_, wf2_, bf2_,
             out_ref, sums_lo, sums_hi, cnts):
        i = pl.program_id(0)

        @pl.when(i == 0)
        def _():
            sums_lo[...] = jnp.zeros_like(sums_lo)
            sums_hi[...] = jnp.zeros_like(sums_hi)
            cnts[...] = jnp.zeros_like(cnts)

        bv = b_[...]
        h2lo = alo_[...] + bv[:, :H // 2]
        h2hi = ahi_[...] + bv[:, H // 2:]
        gids = lax.broadcasted_iota(jnp.int32, (G, blk), 0)
        oh = (gids == bt[...].reshape(1, blk)).astype(jnp.float32)
        hp = lax.Precision.HIGHEST
        sums_lo[...] += jnp.dot(oh, h2lo, preferred_element_type=jnp.float32,
                                precision=hp)
        sums_hi[...] += jnp.dot(oh, h2hi, preferred_element_type=jnp.float32,
                                precision=hp)
        cnts[...] += jnp.broadcast_to(jnp.sum(oh, axis=1, keepdims=True),
                                      (G, 128))

        @pl.when(i == nb - 1)
        def _():
            cc = jnp.broadcast_to(jnp.maximum(cnts[...][:, :1], 1.0),
                                  (G, H // 2))
            plo = sums_lo[...] / cc
            phi = sums_hi[...] / cc
            wf1v = wf1_[...]
            z = jnp.maximum(
                jnp.dot(plo, wf1v[:H // 2], preferred_element_type=jnp.float32,
                     precision=lax.Precision.HIGHEST)
                + jnp.dot(phi, wf1v[H // 2:],
                          preferred_element_type=jnp.float32,
                     precision=lax.Precision.HIGHEST)
                + bf1_[...], 0.0)
            out_ref[...] = (jnp.dot(z, wf2_[...],
                                    preferred_element_type=jnp.float32,
                     precision=lax.Precision.HIGHEST)
                            + bf2_[...])

    return pl.pallas_call(
        body,
        grid=(nb,),
        in_specs=[
            pl.BlockSpec((blk, H // 2), lambda i: (i, 0)),
            pl.BlockSpec((blk, H // 2), lambda i: (i, 0)),
            pl.BlockSpec((1, H), lambda i: (0, 0)),
            pl.BlockSpec((1, 1, blk), lambda i: (i, 0, 0)),
            pl.BlockSpec((H, 64), lambda i: (0, 0)),
            pl.BlockSpec((1, 64), lambda i: (0, 0)),
            pl.BlockSpec((64, 128), lambda i: (0, 0)),
            pl.BlockSpec((1, 128), lambda i: (0, 0)),
        ],
        out_specs=pl.BlockSpec((G, 128), lambda i: (0, 0)),
        out_shape=jax.ShapeDtypeStruct((G, 128), jnp.float32),
        scratch_shapes=[pltpu.VMEM((G, H // 2), jnp.float32),
                        pltpu.VMEM((G, H // 2), jnp.float32),
                        pltpu.VMEM((G, 128), jnp.float32)],
    )(alo, ahi, b2r, batch2d, wf1, bf1r, wf2p, bf2r)


def _dbg_agg_jnp(f_lo, f_hi, srcp, dstp, ewp, dinv):
    f = jnp.concatenate([f_lo, f_hi], axis=1)
    nrm = dinv[srcp] * ewp * dinv[dstp]
    msg = f[srcp] * nrm[:, None]
    out = jnp.zeros_like(f).at[dstp].add(msg)
    out = out + (dinv * dinv)[:, None] * f
    return out[:, :H // 2], out[:, H // 2:]


def kernel(x, edge_index, edge_weight, batch, W1, b1, W2, b2,
           Wf1, bf1, Wf2, bf2):
    src = edge_index[0]
    dst = edge_index[1]
    srcp = jnp.pad(src, (0, EP - E))
    dstp = jnp.pad(dst, (0, EP - E))
    src2 = srcp.reshape(EP // _EC2, _EC2)
    dst2 = dstp.reshape(EP // _EC2, _EC2)
    ewp = jnp.pad(edge_weight, (0, EP - E))
    xp = jnp.pad(x, ((0, NP - N), (0, 0)))
    batch2d = jnp.pad(batch, (0, NP - N), constant_values=-1).reshape(
        NP // 256, 1, 256)

    degp = _deg_call(dstp, ewp)
    dinv = _dinv_call(degp)

    ax0, ax1 = _agg1_call(xp, src2, dst2, ewp, dinv)
    agg1_lo, agg1_hi = _tc_mm1(ax0, ax1, W1)
    hh_lo, hh_hi = _tc_mm2(agg1_lo, agg1_hi, b1.reshape(1, H), W2)
    agg2_lo, agg2_hi = _agg2_call(hh_lo, hh_hi, src2, dst2, ewp, dinv)

    outp = _tc_head(agg2_lo, agg2_hi, b2.reshape(1, H), batch2d,
                    Wf1, bf1.reshape(1, 64),
                    jnp.pad(Wf2, ((0, 0), (0, 125))),
                    jnp.pad(bf2, (0, 125)).reshape(1, 128))
    return outp[:, :3]


# speedup vs baseline: 8.7188x; 1.0065x over previous
import functools

import jax
import jax.numpy as jnp
from jax import lax
from jax.experimental import pallas as pl
from jax.experimental.pallas import tpu as pltpu
from jax.experimental.pallas import tpu_sc as plsc

N = 10000
NP = 10240
E = 320000
EP = 327680
D = 128
H = 256
G = 64

_MESH = plsc.VectorSubcoreMesh(core_axis_name="c", subcore_axis_name="s")

_EC = 2048
_EC2 = 128


def _deg_body(dst_hbm, ew_hbm, out_hbm, deg_v, idx_v, ewc_v, stage_sh):
    c = lax.axis_index("c")
    s = lax.axis_index("s")
    w = s * 2 + c

    def zero(i, _):
        deg_v[pl.ds(i * 16, 16)] = jnp.zeros((16,), jnp.float32)
        return 0

    lax.fori_loop(0, NP // 16, zero, 0)

    tile_base = pl.multiple_of(w * (EP // 32), 8)
    for k in range(EP // 32 // _EC):
        base = pl.multiple_of(tile_base + k * _EC, 8)
        pltpu.sync_copy(dst_hbm.at[pl.ds(base, _EC)], idx_v)
        pltpu.sync_copy(ew_hbm.at[pl.ds(base, _EC)], ewc_v)

        def scat(j, _):
            sl = pl.ds(j * 16, 16)
            plsc.addupdate_scatter(deg_v, [idx_v[sl]], ewc_v[sl])
            return 0

        lax.fori_loop(0, _EC // 16, scat, 0)

    pltpu.sync_copy(deg_v, stage_sh.at[pl.ds(s * NP, NP)])
    plsc.subcore_barrier()

    col = pl.multiple_of(s * (NP // 16), 8)
    lax.fori_loop(0, NP // 16 // 16, zero, 0)
    for r in range(16):
        pltpu.sync_copy(stage_sh.at[pl.ds(r * NP + col, NP // 16)],
                        ewc_v.at[pl.ds(0, NP // 16)])

        def acc(j, _):
            sl = pl.ds(j * 16, 16)
            deg_v[sl] = deg_v[sl] + ewc_v[sl]
            return 0

        lax.fori_loop(0, NP // 16 // 16, acc, 0)
    pltpu.sync_copy(deg_v.at[pl.ds(0, NP // 16)],
                    out_hbm.at[pl.ds(c * NP + col, NP // 16)])


_deg_call = functools.partial(
    pl.kernel,
    out_type=jax.ShapeDtypeStruct((2 * NP,), jnp.float32),
    mesh=_MESH,
    compiler_params=pltpu.CompilerParams(needs_layout_passes=False),
    scratch_types=[
        pltpu.VMEM((NP,), jnp.float32),
        pltpu.VMEM((_EC,), jnp.int32),
        pltpu.VMEM((_EC,), jnp.float32),
        pltpu.VMEM_SHARED((16 * NP,), jnp.float32),
    ],
)(_deg_body)


def _dinv_body(degp_hbm, dinv_hbm, a_v, b_v):
    c = lax.axis_index("c")
    s = lax.axis_index("s")
    w = s * 2 + c
    nt = NP // 32
    base = pl.multiple_of(w * nt, 8)
    pltpu.sync_copy(degp_hbm.at[pl.ds(base, nt)], a_v)
    pltpu.sync_copy(degp_hbm.at[pl.ds(NP + base, nt)], b_v)

    def body(j, _):
        sl = pl.ds(j * 16, 16)
        d = a_v[sl] + b_v[sl] + 1.0
        i = plsc.bitcast(d, jnp.int32)
        i = jnp.int32(0x5F3759DF) - (i >> 1)
        y = plsc.bitcast(i, jnp.float32)
        for _unused in range(3):
            y = y * (1.5 - 0.5 * d * y * y)
        a_v[sl] = y
        return 0

    lax.fori_loop(0, nt // 16, body, 0)
    pltpu.sync_copy(a_v, dinv_hbm.at[pl.ds(base, nt)])


_dinv_call = functools.partial(
    pl.kernel,
    out_type=jax.ShapeDtypeStruct((NP,), jnp.float32),
    mesh=_MESH,
    compiler_params=pltpu.CompilerParams(needs_layout_passes=False),
    scratch_types=[
        pltpu.VMEM((NP // 32,), jnp.float32),
        pltpu.VMEM((NP // 32,), jnp.float32),
    ],
)(_dinv_body)


_RB = 128


def _zero_rows(rows_v, nrows, width):
    def z(r, _):
        for j in range(width // 16):
            rows_v[r, pl.ds(j * 16, 16)] = jnp.zeros((16,), jnp.float32)
        return 0

    lax.fori_loop(0, nrows, z, 0)


def _init_self_loop(f_hbm, dinv_v, rows_v, acc_sh, s, width):
    for kb in range(640 // _RB):
        rbase = pl.multiple_of(s * 640 + kb * _RB, 8)
        pltpu.sync_copy(f_hbm.at[pl.ds(rbase, _RB)], rows_v.at[pl.ds(0, _RB)])

        def init_scale(jj, _):
            dv = dinv_v[pl.ds(rbase + jj * 16, 16)]
            dv2 = dv * dv
            for r in range(16):
                s2 = dv2[r]
                row = jj * 16 + r
                for j in range(width // 16):
                    sl = pl.ds(j * 16, 16)
                    rows_v[row, sl] = rows_v[row, sl] * s2
            return 0

        lax.fori_loop(0, _RB // 16, init_scale, 0)
        pltpu.sync_copy(rows_v.at[pl.ds(0, _RB)], acc_sh.at[pl.ds(rbase, _RB)])


_SCN = 8


def _edge_loop(f_hbm, src2_hbm, dst2_hbm, ew_hbm, dinv_v,
               src2_v, dst2_v, nrm_v, rows_a, rows_b,
               sem_ga, sem_gb, sem_sa, sem_sb, acc_sh, crow0, nch, width):
    nsc = nch // _SCN

    def scale(buf, kk):
        def scale16(jj, _):
            nv = nrm_v[pl.ds(kk * _EC2 + jj * 16, 16)]
            for r in range(16):
                sc = nv[r]
                row = jj * 16 + r
                for j in range(width // 16):
                    sl = pl.ds(j * 16, 16)
                    buf[row, sl] = buf[row, sl] * sc
            return 0

        lax.fori_loop(0, _EC2 // 16, scale16, 0)

    def gather(kk, buf, sem):
        pltpu.async_copy(f_hbm.at[src2_v.at[kk]], buf, sem)

    def scatter(kk, buf, sem):
        pltpu.async_copy(buf, acc_sh.at[dst2_v.at[kk]], sem, add=True)

    def wait_gather(buf, sem):
        pltpu.make_async_copy(f_hbm.at[src2_v.at[0]], buf, sem).wait()

    def wait_scatter(buf, sem):
        pltpu.make_async_copy(buf, acc_sh.at[dst2_v.at[0]], sem).wait()

    def superblock(u, _):
        crow = pl.multiple_of(crow0 + u * _SCN, 8)
        ebase = pl.multiple_of((crow0 + u * _SCN) * _EC2, 8)
        pltpu.sync_copy(src2_hbm.at[pl.ds(crow, _SCN)], src2_v)
        pltpu.sync_copy(dst2_hbm.at[pl.ds(crow, _SCN)], dst2_v)
        pltpu.sync_copy(ew_hbm.at[pl.ds(ebase, _SCN * _EC2)], nrm_v)

        gather(0, rows_a, sem_ga)
        gather(1, rows_b, sem_gb)

        def nrm_row(r, _):
            for j in range(_EC2 // 16):
                sl = pl.ds(j * 16, 16)
                fl = pl.ds(r * _EC2 + j * 16, 16)
                n1 = plsc.load_gather(dinv_v, [src2_v[r, sl]])
                n2 = plsc.load_gather(dinv_v, [dst2_v[r, sl]])
                nrm_v[fl] = n1 * nrm_v[fl] * n2
            return 0

        lax.fori_loop(0, _SCN, nrm_row, 0)

        def pipe(t, _):
            ka = 2 * t
            kb = 2 * t + 1

            @pl.when(t > 0)
            def _():
                wait_scatter(rows_b, sem_sb)
                gather(kb, rows_b, sem_gb)
            wait_gather(rows_a, sem_ga)
            scale(rows_a, ka)
            scatter(ka, rows_a, sem_sa)

            @pl.when(t < _SCN // 2 - 1)
            def _():
                wait_scatter(rows_a, sem_sa)
                gather(ka + 2, rows_a, sem_ga)

            wait_gather(rows_b, sem_gb)
            scale(rows_b, kb)
            scatter(kb, rows_b, sem_sb)
            return 0

        lax.fori_loop(0, _SCN // 2, pipe, 0)
        wait_scatter(rows_a, sem_sa)
        wait_scatter(rows_b, sem_sb)
        return 0

    lax.fori_loop(0, nsc, superblock, 0)


def _writeout(o_hbm, rows_v, acc_sh, s):
    for kb in range(640 // _RB):
        rbase = pl.multiple_of(s * 640 + kb * _RB, 8)
        pltpu.sync_copy(acc_sh.at[pl.ds(rbase, _RB)], rows_v.at[pl.ds(0, _RB)])
        pltpu.sync_copy(rows_v.at[pl.ds(0, _RB)], o_hbm.at[pl.ds(rbase, _RB)])


def _agg1_body(f_hbm, src2_hbm, dst2_hbm, ew_hbm, dinv_hbm,
               o0_hbm, o1_hbm,
               dinv_v, src2_v, dst2_v, nrm_v, rows_a, rows_b,
               sem_ga, sem_gb, sem_sa, sem_sb, acc_sh):
    c = lax.axis_index("c")
    s = lax.axis_index("s")
    pltpu.sync_copy(dinv_hbm, dinv_v)

    @pl.when(c == 0)
    def _():
        _init_self_loop(f_hbm, dinv_v, rows_a, acc_sh, s, D)

    @pl.when(c == 1)
    def _():
        _zero_rows(rows_a, _RB, D)
        for kb in range(640 // _RB):
            rbase = pl.multiple_of(s * 640 + kb * _RB, 8)
            pltpu.sync_copy(rows_a.at[pl.ds(0, _RB)],
                            acc_sh.at[pl.ds(rbase, _RB)])

    plsc.subcore_barrier()
    nch1 = EP // 32 // _EC2
    _edge_loop(f_hbm, src2_hbm, dst2_hbm, ew_hbm, dinv_v,
               src2_v, dst2_v, nrm_v, rows_a, rows_b,
               sem_ga, sem_gb, sem_sa, sem_sb, acc_sh,
               (c * 16 + s) * nch1, nch1, D)
    plsc.subcore_barrier()

    @pl.when(c == 0)
    def _():
        _writeout(o0_hbm, rows_a, acc_sh, s)

    @pl.when(c == 1)
    def _():
        _writeout(o1_hbm, rows_a, acc_sh, s)


_agg1_call = functools.partial(
    pl.kernel,
    out_type=(jax.ShapeDtypeStruct((NP, D), jnp.float32),
              jax.ShapeDtypeStruct((NP, D), jnp.float32)),
    mesh=_MESH,
    compiler_params=pltpu.CompilerParams(needs_layout_passes=False),
    scratch_types=[
        pltpu.VMEM((NP,), jnp.float32),
        pltpu.VMEM((_SCN, _EC2), jnp.int32),
        pltpu.VMEM((_SCN, _EC2), jnp.int32),
        pltpu.VMEM((_SCN * _EC2,), jnp.float32),
        pltpu.VMEM((_EC2, D), jnp.float32),
        pltpu.VMEM((_EC2, D), jnp.float32),
        pltpu.SemaphoreType.DMA,
        pltpu.SemaphoreType.DMA,
        pltpu.SemaphoreType.DMA,
        pltpu.SemaphoreType.DMA,
        pltpu.VMEM_SHARED((NP, D), jnp.float32),
    ],
)(_agg1_body)


def _agg2_body(flo_hbm, fhi_hbm, src2_hbm, dst2_hbm, ew_hbm, dinv_hbm,
               olo_hbm, ohi_hbm,
               dinv_v, src2_v, dst2_v, nrm_v, rows_a, rows_b,
               sem_ga, sem_gb, sem_sa, sem_sb, acc_sh):
    c = lax.axis_index("c")
    s = lax.axis_index("s")
    F2 = H // 2
    pltpu.sync_copy(dinv_hbm, dinv_v)

    def run(f_hbm, o_hbm):
        _init_self_loop(f_hbm, dinv_v, rows_a, acc_sh, s, F2)
        plsc.subcore_barrier()
        _edge_loop(f_hbm, src2_hbm, dst2_hbm, ew_hbm, dinv_v,
                   src2_v, dst2_v, nrm_v, rows_a, rows_b,
                   sem_ga, sem_gb, sem_sa, sem_sb, acc_sh,
                   s * (EP // 16 // _EC2), EP // 16 // _EC2, F2)
        plsc.subcore_barrier()
        _writeout(o_hbm, rows_a, acc_sh, s)

    @pl.when(c == 0)
    def _c0():
        run(flo_hbm, olo_hbm)

    @pl.when(c == 1)
    def _c1():
        run(fhi_hbm, ohi_hbm)


_agg2_call = functools.partial(
    pl.kernel,
    out_type=(jax.ShapeDtypeStruct((NP, H // 2), jnp.float32),
              jax.ShapeDtypeStruct((NP, H // 2), jnp.float32)),
    mesh=_MESH,
    compiler_params=pltpu.CompilerParams(needs_layout_passes=False),
    scratch_types=[
        pltpu.VMEM((NP,), jnp.float32),
        pltpu.VMEM((_SCN, _EC2), jnp.int32),
        pltpu.VMEM((_SCN, _EC2), jnp.int32),
        pltpu.VMEM((_SCN * _EC2,), jnp.float32),
        pltpu.VMEM((_EC2, H // 2), jnp.float32),
        pltpu.VMEM((_EC2, H // 2), jnp.float32),
        pltpu.SemaphoreType.DMA,
        pltpu.SemaphoreType.DMA,
        pltpu.SemaphoreType.DMA,
        pltpu.SemaphoreType.DMA,
        pltpu.VMEM_SHARED((NP, H // 2), jnp.float32),
    ],
)(_agg2_body)


def _tc_mm1(a0, a1, w1):
    blk = 256

    def body(a0_, a1_, w_, olo, ohi):
        h = jnp.dot(a0_[...] + a1_[...], w_[...],
                    preferred_element_type=jnp.float32,
                    precision=lax.Precision.HIGHEST)
        olo[...] = h[:, :H // 2]
        ohi[...] = h[:, H // 2:]

    return pl.pallas_call(
        body,
        grid=(NP // blk,),
        in_specs=[
            pl.BlockSpec((blk, D), lambda i: (i, 0)),
            pl.BlockSpec((blk, D), lambda i: (i, 0)),
            pl.BlockSpec((D, H), lambda i: (0, 0)),
        ],
        out_specs=[pl.BlockSpec((blk, H // 2), lambda i: (i, 0)),
                   pl.BlockSpec((blk, H // 2), lambda i: (i, 0))],
        out_shape=[jax.ShapeDtypeStruct((NP, H // 2), jnp.float32),
                   jax.ShapeDtypeStruct((NP, H // 2), jnp.float32)],
    )(a0, a1, w1)


def _tc_mm2(alo, ahi, b1r, w2):
    blk = 256

    def body(alo_, ahi_, b_, w_, olo, ohi):
        bv = b_[...]
        wv = w_[...]
        h1lo = jnp.maximum(alo_[...] + bv[:, :H // 2], 0.0)
        h1hi = jnp.maximum(ahi_[...] + bv[:, H // 2:], 0.0)
        h = (jnp.dot(h1lo, wv[:H // 2], preferred_element_type=jnp.float32,
                     precision=lax.Precision.HIGHEST)
             + jnp.dot(h1hi, wv[H // 2:], preferred_element_type=jnp.float32,
                     precision=lax.Precision.HIGHEST))
        olo[...] = h[:, :H // 2]
        ohi[...] = h[:, H // 2:]

    return pl.pallas_call(
        body,
        grid=(NP // blk,),
        in_specs=[
            pl.BlockSpec((blk, H // 2), lambda i: (i, 0)),
            pl.BlockSpec((blk, H // 2), lambda i: (i, 0)),
            pl.BlockSpec((1, H), lambda i: (0, 0)),
            pl.BlockSpec((H, H), lambda i: (0, 0)),
        ],
        out_specs=[pl.BlockSpec((blk, H // 2), lambda i: (i, 0)),
                   pl.BlockSpec((blk, H // 2), lambda i: (i, 0))],
        out_shape=[jax.ShapeDtypeStruct((NP, H // 2), jnp.float32),
                   jax.ShapeDtypeStruct((NP, H // 2), jnp.float32)],
    )(alo, ahi, b1r, w2)


def _tc_head(alo, ahi, b2r, batch2d, wf1, bf1r, wf2p, bf2r):
    blk = 256
    nb = NP // blk

    def body(alo_, ahi_, b_, bt, wf1_, bf1_, wf2_, bf2_,
             out_ref, sums_lo, sums_hi, cnts):
        i = pl.program_id(0)

        @pl.when(i == 0)
        def _():
            sums_lo[...] = jnp.zeros_like(sums_lo)
            sums_hi[...] = jnp.zeros_like(sums_hi)
            cnts[...] = jnp.zeros_like(cnts)

        bv = b_[...]
        h2lo = alo_[...] + bv[:, :H // 2]
        h2hi = ahi_[...] + bv[:, H // 2:]
        gids = lax.broadcasted_iota(jnp.int32, (G, blk), 0)
        oh = (gids == bt[...].reshape(1, blk)).astype(jnp.float32)
        hp = lax.Precision.HIGHEST
        sums_lo[...] += jnp.dot(oh, h2lo, preferred_element_type=jnp.float32,
                                precision=hp)
        sums_hi[...] += jnp.dot(oh, h2hi, preferred_element_type=jnp.float32,
                                precision=hp)
        cnts[...] += jnp.broadcast_to(jnp.sum(oh, axis=1, keepdims=True),
                                      (G, 128))

        @pl.when(i == nb - 1)
        def _():
            cc = jnp.broadcast_to(jnp.maximum(cnts[...][:, :1], 1.0),
                                  (G, H // 2))
            plo = sums_lo[...] / cc
            phi = sums_hi[...] / cc
            wf1v = wf1_[...]
            z = jnp.maximum(
                jnp.dot(plo, wf1v[:H // 2], preferred_element_type=jnp.float32,
                     precision=lax.Precision.HIGHEST)
                + jnp.dot(phi, wf1v[H // 2:],
                          preferred_element_type=jnp.float32,
                     precision=lax.Precision.HIGHEST)
                + bf1_[...], 0.0)
            out_ref[...] = (jnp.dot(z, wf2_[...],
                                    preferred_element_type=jnp.float32,
                     precision=lax.Precision.HIGHEST)
                            + bf2_[...])

    return pl.pallas_call(
        body,
        grid=(nb,),
        in_specs=[
            pl.BlockSpec((blk, H // 2), lambda i: (i, 0)),
            pl.BlockSpec((blk, H // 2), lambda i: (i, 0)),
            pl.BlockSpec((1, H), lambda i: (0, 0)),
            pl.BlockSpec((1, 1, blk), lambda i: (i, 0, 0)),
            pl.BlockSpec((H, 64), lambda i: (0, 0)),
            pl.BlockSpec((1, 64), lambda i: (0, 0)),
            pl.BlockSpec((64, 128), lambda i: (0, 0)),
            pl.BlockSpec((1, 128), lambda i: (0, 0)),
        ],
        out_specs=pl.BlockSpec((G, 128), lambda i: (0, 0)),
        out_shape=jax.ShapeDtypeStruct((G, 128), jnp.float32),
        scratch_shapes=[pltpu.VMEM((G, H // 2), jnp.float32),
                        pltpu.VMEM((G, H // 2), jnp.float32),
                        pltpu.VMEM((G, 128), jnp.float32)],
    )(alo, ahi, b2r, batch2d, wf1, bf1r, wf2p, bf2r)


def kernel(x, edge_index, edge_weight, batch, W1, b1, W2, b2,
           Wf1, bf1, Wf2, bf2):
    src = edge_index[0]
    dst = edge_index[1]
    srcp = jnp.pad(src, (0, EP - E))
    dstp = jnp.pad(dst, (0, EP - E))
    src2 = srcp.reshape(EP // _EC2, _EC2)
    dst2 = dstp.reshape(EP // _EC2, _EC2)
    ewp = jnp.pad(edge_weight, (0, EP - E))
    xp = jnp.pad(x, ((0, NP - N), (0, 0)))
    batch2d = jnp.pad(batch, (0, NP - N), constant_values=-1).reshape(
        NP // 256, 1, 256)

    degp = _deg_call(dstp, ewp)
    dinv = _dinv_call(degp)

    ax0, ax1 = _agg1_call(xp, src2, dst2, ewp, dinv)
    agg1_lo, agg1_hi = _tc_mm1(ax0, ax1, W1)
    hh_lo, hh_hi = _tc_mm2(agg1_lo, agg1_hi, b1.reshape(1, H), W2)
    agg2_lo, agg2_hi = _agg2_call(hh_lo, hh_hi, src2, dst2, ewp, dinv)

    outp = _tc_head(agg2_lo, agg2_hi, b2.reshape(1, H), batch2d,
                    Wf1, bf1.reshape(1, 64),
                    jnp.pad(Wf2, ((0, 0), (0, 125))),
                    jnp.pad(bf2, (0, 125)).reshape(1, 128))
    return outp[:, :3]
